# baseline (speedup 1.0000x reference)
import jax
import jax.numpy as jnp
from jax.experimental import pallas as pl


def _add_kernel(x_ref, tab_ref, o_ref):
    o_ref[...] = x_ref[...] + tab_ref[...]


def kernel(x, pos_emb_table):
    batch, seq, dim = x.shape
    blk_s = 128
    grid = (seq // blk_s,)
    return pl.pallas_call(
        _add_kernel,
        grid=grid,
        in_specs=[
            pl.BlockSpec((batch, blk_s, dim), lambda s: (0, s, 0)),
            pl.BlockSpec((blk_s, dim), lambda s: (s, 0)),
        ],
        out_specs=pl.BlockSpec((batch, blk_s, dim), lambda s: (0, s, 0)),
        out_shape=jax.ShapeDtypeStruct(x.shape, x.dtype),
    )(x, pos_emb_table)


# speedup vs baseline: 1.1825x; 1.1825x over previous
import jax
import jax.numpy as jnp
from jax.experimental import pallas as pl


def _add_kernel(x_ref, tab_ref, o_ref):
    o_ref[...] = x_ref[...] + tab_ref[...]


def kernel(x, pos_emb_table):
    batch, seq, dim = x.shape
    blk_s = 512
    grid = (seq // blk_s,)
    return pl.pallas_call(
        _add_kernel,
        grid=grid,
        in_specs=[
            pl.BlockSpec((batch, blk_s, dim), lambda s: (0, s, 0)),
            pl.BlockSpec((blk_s, dim), lambda s: (s, 0)),
        ],
        out_specs=pl.BlockSpec((batch, blk_s, dim), lambda s: (0, s, 0)),
        out_shape=jax.ShapeDtypeStruct(x.shape, x.dtype),
    )(x, pos_emb_table)
